# baseline (speedup 1.0000x reference)
import numpy as np
import jax
import jax.numpy as jnp
from jax.experimental import pallas as pl
from jax.experimental.pallas import tpu as pltpu

_IN = 64
_GRID = (180, 200, 5)
_N = _GRID[0] * _GRID[1] * _GRID[2]
_A = 6
_BBOX = 7
_NCLS = 4
_CLS_C = _A * _NCLS
_REG_C = _A * _BBOX
_TN = 2048

_PC_RANGE = np.array([0.0, -40.0, -4.0, 72.0, 40.0, 4.0], dtype=np.float32)
_ANCHOR_SIZES = np.array([
    [3.9, 1.6, 1.56, 0.0],
    [3.9, 1.6, 1.56, 1.5707963],
    [0.8, 0.6, 1.73, 0.0],
    [0.8, 0.6, 1.73, 1.5707963],
    [1.76, 0.6, 1.73, 0.0],
    [1.76, 0.6, 1.73, 1.5707963],
], dtype=np.float32)
_STRIDE = (_PC_RANGE[3:] - _PC_RANGE[:3]) / np.array(_GRID, dtype=np.float32)

_LANE_K = np.arange(_REG_C) % _BBOX
_LANE_A = np.arange(_REG_C) // _BBOX
_MULT = np.where(_LANE_K < 6, _ANCHOR_SIZES[_LANE_A, _LANE_K % 3], 0.0).astype(np.float32)
_COSANG = np.where(_LANE_K == 6, np.cos(_ANCHOR_SIZES[_LANE_A, 3]), 0.0).astype(np.float32)
_IS_XYZ = (_LANE_K < 3).astype(np.float32)
_IS_ANG = (_LANE_K == 6).astype(np.float32)
_SEL_X = (_LANE_K == 0).astype(np.float32)
_SEL_Y = (_LANE_K == 1).astype(np.float32)
_SEL_Z = (_LANE_K == 2).astype(np.float32)
_DECODE_TAB = np.stack(
    [_MULT, _COSANG, _IS_XYZ, _IS_ANG, _SEL_X, _SEL_Y, _SEL_Z,
     np.zeros(_REG_C, np.float32)], axis=0)


def _head_kernel(x_ref, w1c_ref, b1c_ref, w2c_ref, b2c_ref,
                 w1r_ref, b1r_ref, w2r_ref, b2r_ref, tab_ref,
                 cls_ref, box_ref):
    xt = x_ref[0].T

    hc = jnp.maximum(
        jnp.dot(xt, w1c_ref[...], preferred_element_type=jnp.float32)
        + b1c_ref[...], 0.0)
    cls_ref[0] = (jnp.dot(hc, w2c_ref[...], preferred_element_type=jnp.float32)
                  + b2c_ref[...])

    hr = jnp.maximum(
        jnp.dot(xt, w1r_ref[...], preferred_element_type=jnp.float32)
        + b1r_ref[...], 0.0)
    off = (jnp.dot(hr, w2r_ref[...], preferred_element_type=jnp.float32)
           + b2r_ref[...])

    row = pl.program_id(1) * _TN + jax.lax.broadcasted_iota(jnp.int32, (_TN, 1), 0)
    iz = (row % _GRID[2]).astype(jnp.float32)
    iy = ((row // _GRID[2]) % _GRID[1]).astype(jnp.float32)
    ix = (row // (_GRID[1] * _GRID[2])).astype(jnp.float32)
    cx = (_PC_RANGE[0] + _STRIDE[0] * ix) + np.float32(_STRIDE[0] / 2.0)
    cy = (_PC_RANGE[1] + _STRIDE[1] * iy) + np.float32(_STRIDE[1] / 2.0)
    cz = (_PC_RANGE[2] + _STRIDE[2] * iz) + np.float32(_STRIDE[2] / 2.0)

    mult = tab_ref[0:1, :]
    coord = (cx * tab_ref[4:5, :] + cy * tab_ref[5:6, :] + cz * tab_ref[6:7, :])
    xyz_v = off * mult + coord
    dim_v = jnp.exp(off) * mult
    c = jax.nn.sigmoid(off) * tab_ref[1:2, :]
    ang_v = jnp.arctan2(c, jnp.sqrt(1.0 - c * c))
    box_ref[0] = jnp.where(tab_ref[3:4, :] > 0.5, ang_v,
                           jnp.where(tab_ref[2:3, :] > 0.5, xyz_v, dim_v))


def kernel(x, W1c, b1c, gc, bec, mc, vc, W2c, b2c, W1r, b1r, gr, ber, mr, vr, W2r, b2r):
    B = x.shape[0]
    xr = x.reshape(B, _IN, _N)

    sc = gc * jax.lax.rsqrt(vc + 1e-5)
    w1c_eff = W1c.T * sc[None, :]
    b1c_eff = (b1c * sc + (bec - mc * sc))[None, :]
    sr = gr * jax.lax.rsqrt(vr + 1e-5)
    w1r_eff = W1r.T * sr[None, :]
    b1r_eff = (b1r * sr + (ber - mr * sr))[None, :]

    grid = (B, pl.cdiv(_N, _TN))
    full = lambda shape: pl.BlockSpec(shape, lambda b, t: (0,) * len(shape))
    cls_out, box_out = pl.pallas_call(
        _head_kernel,
        grid=grid,
        in_specs=[
            pl.BlockSpec((1, _IN, _TN), lambda b, t: (b, 0, t)),
            full((_IN, _IN)), full((1, _IN)), full((_IN, _CLS_C)), full((1, _CLS_C)),
            full((_IN, _IN)), full((1, _IN)), full((_IN, _REG_C)), full((1, _REG_C)),
            full((8, _REG_C)),
        ],
        out_specs=[
            pl.BlockSpec((1, _TN, _CLS_C), lambda b, t: (b, t, 0)),
            pl.BlockSpec((1, _TN, _REG_C), lambda b, t: (b, t, 0)),
        ],
        out_shape=[
            jax.ShapeDtypeStruct((B, _N, _CLS_C), jnp.float32),
            jax.ShapeDtypeStruct((B, _N, _REG_C), jnp.float32),
        ],
        compiler_params=pltpu.CompilerParams(
            dimension_semantics=("parallel", "parallel")),
    )(xr, w1c_eff, b1c_eff, W2c.T, b2c[None, :],
      w1r_eff, b1r_eff, W2r.T, b2r[None, :], jnp.asarray(_DECODE_TAB))

    cls_out = cls_out.reshape((B,) + _GRID + (_CLS_C,))
    box_out = box_out.reshape((B,) + _GRID + (_REG_C,))
    return cls_out, box_out


# speedup vs baseline: 3.1166x; 3.1166x over previous
import numpy as np
import jax
import jax.numpy as jnp
from jax.experimental import pallas as pl
from jax.experimental.pallas import tpu as pltpu

_IN = 64
_GX, _GY, _GZ = 180, 200, 5
_A = 6
_BBOX = 7
_CLS_C = _A * 4
_REG_C = _A * _BBOX
_XB = 12

_PC_RANGE = np.array([0.0, -40.0, -4.0, 72.0, 40.0, 4.0], dtype=np.float32)
_ANCHOR_SIZES = np.array([
    [3.9, 1.6, 1.56, 0.0],
    [3.9, 1.6, 1.56, 1.5707963],
    [0.8, 0.6, 1.73, 0.0],
    [0.8, 0.6, 1.73, 1.5707963],
    [1.76, 0.6, 1.73, 0.0],
    [1.76, 0.6, 1.73, 1.5707963],
], dtype=np.float32)
_STRIDE = (_PC_RANGE[3:] - _PC_RANGE[:3]) / np.array([_GX, _GY, _GZ], np.float32)

_PERM = np.array([7 * a + k for a in range(_A) for k in range(6)]
                 + [7 * a + 6 for a in range(_A)], dtype=np.int32)
_KM = np.tile(np.arange(6), _A)
_AM = np.repeat(np.arange(_A), 6)
_MULT = _ANCHOR_SIZES[_AM, _KM % 3].astype(np.float32)[:, None]
_IS_XYZ = (_KM < 3).astype(np.float32)[:, None]
_SEL_X = (_KM == 0).astype(np.float32)[:, None]
_SEL_Y = (_KM == 1).astype(np.float32)[:, None]
_SEL_Z = (_KM == 2).astype(np.float32)[:, None]
_COS = np.cos(_ANCHOR_SIZES[:, 3]).astype(np.float32)[:, None]
_TAB = np.concatenate([_MULT, _IS_XYZ, _SEL_X, _SEL_Y, _SEL_Z], axis=1)


def _head_kernel(x_ref, w1c_ref, b1c_ref, w2c_ref, b2c_ref,
                 w1r_ref, b1r_ref, w2r_ref, b2r_ref, tab_ref, cos_ref,
                 cls_ref, box_ref):
    zi = pl.program_id(2)
    xrow0 = pl.program_id(1) * _XB
    cz = (_PC_RANGE[2] + _STRIDE[2] * zi.astype(jnp.float32)) + np.float32(_STRIDE[2] / 2.0)
    iy = jax.lax.broadcasted_iota(jnp.int32, (1, _GY), 1).astype(jnp.float32)
    cy = (_PC_RANGE[1] + _STRIDE[1] * iy) + np.float32(_STRIDE[1] / 2.0)

    mult = tab_ref[:, 0:1]
    is_xyz = tab_ref[:, 1:2] > 0.5
    coord_yz = cy * tab_ref[:, 3:4] + cz * tab_ref[:, 4:5]
    sel_x = tab_ref[:, 2:3]
    cosang = cos_ref[...]

    for i in range(_XB):
        xb = x_ref[0, i, 0]
        hc = jnp.maximum(
            jnp.dot(w1c_ref[...], xb, preferred_element_type=jnp.float32)
            + b1c_ref[...], 0.0)
        cls_ref[0, i, 0] = (
            jnp.dot(w2c_ref[...], hc, preferred_element_type=jnp.float32)
            + b2c_ref[...])

        hr = jnp.maximum(
            jnp.dot(w1r_ref[...], xb, preferred_element_type=jnp.float32)
            + b1r_ref[...], 0.0)
        off = (jnp.dot(w2r_ref[...], hr, preferred_element_type=jnp.float32)
               + b2r_ref[...])

        cx = (_PC_RANGE[0] + _STRIDE[0] * (xrow0 + i).astype(jnp.float32)) \
            + np.float32(_STRIDE[0] / 2.0)
        om = off[0:36]
        dec_main = jnp.where(is_xyz, om * mult + (coord_yz + cx * sel_x),
                             jnp.exp(om) * mult)
        c = jax.nn.sigmoid(off[36:42]) * cosang
        ang = jnp.arctan2(c, jnp.sqrt(1.0 - c * c))

        pieces = []
        for a in range(_A):
            pieces.append(dec_main[6 * a:6 * a + 6])
            pieces.append(ang[a:a + 1])
        box_ref[0, i, 0] = jnp.concatenate(pieces, axis=0)


def kernel(x, W1c, b1c, gc, bec, mc, vc, W2c, b2c, W1r, b1r, gr, ber, mr, vr, W2r, b2r):
    B = x.shape[0]
    xt = jnp.transpose(x, (0, 2, 4, 1, 3))

    sc = gc * jax.lax.rsqrt(vc + 1e-5)
    w1c_eff = W1c * sc[:, None]
    b1c_eff = (b1c * sc + (bec - mc * sc))[:, None]
    sr = gr * jax.lax.rsqrt(vr + 1e-5)
    w1r_eff = W1r * sr[:, None]
    b1r_eff = (b1r * sr + (ber - mr * sr))[:, None]
    perm = jnp.asarray(_PERM)

    grid = (B, _GX // _XB, _GZ)
    full = lambda shape: pl.BlockSpec(shape, lambda b, xi, zi: (0,) * len(shape))
    cls_t, box_t = pl.pallas_call(
        _head_kernel,
        grid=grid,
        in_specs=[
            pl.BlockSpec((1, _XB, 1, _IN, _GY), lambda b, xi, zi: (b, xi, zi, 0, 0)),
            full((_IN, _IN)), full((_IN, 1)), full((_CLS_C, _IN)), full((_CLS_C, 1)),
            full((_IN, _IN)), full((_IN, 1)), full((_REG_C, _IN)), full((_REG_C, 1)),
            full((36, 5)), full((_A, 1)),
        ],
        out_specs=[
            pl.BlockSpec((1, _XB, 1, _CLS_C, _GY), lambda b, xi, zi: (b, xi, zi, 0, 0)),
            pl.BlockSpec((1, _XB, 1, _REG_C, _GY), lambda b, xi, zi: (b, xi, zi, 0, 0)),
        ],
        out_shape=[
            jax.ShapeDtypeStruct((B, _GX, _GZ, _CLS_C, _GY), jnp.float32),
            jax.ShapeDtypeStruct((B, _GX, _GZ, _REG_C, _GY), jnp.float32),
        ],
        compiler_params=pltpu.CompilerParams(
            dimension_semantics=("parallel", "parallel", "parallel")),
    )(xt, w1c_eff, b1c_eff, W2c, b2c[:, None],
      w1r_eff, b1r_eff, W2r[perm], b2r[perm][:, None],
      jnp.asarray(_TAB), jnp.asarray(_COS))

    cls_out = jnp.transpose(cls_t, (0, 1, 4, 2, 3))
    box_out = jnp.transpose(box_t, (0, 1, 4, 2, 3))
    return cls_out, box_out


# speedup vs baseline: 3.2794x; 1.0522x over previous
import numpy as np
import jax
import jax.numpy as jnp
from jax.experimental import pallas as pl
from jax.experimental.pallas import tpu as pltpu

_IN = 64
_GX, _GY, _GZ = 180, 200, 5
_A = 6
_BBOX = 7
_CLS_C = _A * 4
_REG_C = _A * _BBOX
_XB = 36

_PC_RANGE = np.array([0.0, -40.0, -4.0, 72.0, 40.0, 4.0], dtype=np.float32)
_ANCHOR_SIZES = np.array([
    [3.9, 1.6, 1.56, 0.0],
    [3.9, 1.6, 1.56, 1.5707963],
    [0.8, 0.6, 1.73, 0.0],
    [0.8, 0.6, 1.73, 1.5707963],
    [1.76, 0.6, 1.73, 0.0],
    [1.76, 0.6, 1.73, 1.5707963],
], dtype=np.float32)
_STRIDE = (_PC_RANGE[3:] - _PC_RANGE[:3]) / np.array([_GX, _GY, _GZ], np.float32)

_PERM = np.array([7 * a + k for a in range(_A) for k in range(6)]
                 + [7 * a + 6 for a in range(_A)], dtype=np.int32)
_KM = np.tile(np.arange(6), _A)
_AM = np.repeat(np.arange(_A), 6)
_MULT = _ANCHOR_SIZES[_AM, _KM % 3].astype(np.float32)[:, None]
_IS_XYZ = (_KM < 3).astype(np.float32)[:, None]
_SEL_X = (_KM == 0).astype(np.float32)[:, None]
_SEL_Y = (_KM == 1).astype(np.float32)[:, None]
_SEL_Z = (_KM == 2).astype(np.float32)[:, None]
_COS = np.cos(_ANCHOR_SIZES[:, 3]).astype(np.float32)[:, None]
_TAB = np.concatenate([_MULT, _IS_XYZ, _SEL_X, _SEL_Y, _SEL_Z], axis=1)


def _head_kernel(x_ref, w1c_ref, b1c_ref, w2c_ref, b2c_ref,
                 w1r_ref, b1r_ref, w2r_ref, b2r_ref, tab_ref, cos_ref,
                 cls_ref, box_ref):
    zi = pl.program_id(2)
    xrow0 = pl.program_id(1) * _XB
    cz = (_PC_RANGE[2] + _STRIDE[2] * zi.astype(jnp.float32)) + np.float32(_STRIDE[2] / 2.0)
    iy = jax.lax.broadcasted_iota(jnp.int32, (1, _GY), 1).astype(jnp.float32)
    cy = (_PC_RANGE[1] + _STRIDE[1] * iy) + np.float32(_STRIDE[1] / 2.0)

    mult = tab_ref[:, 0:1]
    is_xyz = tab_ref[:, 1:2] > 0.5
    coord_yz = cy * tab_ref[:, 3:4] + cz * tab_ref[:, 4:5]
    sel_x = tab_ref[:, 2:3]
    cosang = cos_ref[...]

    for i in range(_XB):
        xb = x_ref[0, i, 0]
        hc = jnp.maximum(
            jnp.dot(w1c_ref[...], xb, preferred_element_type=jnp.float32)
            + b1c_ref[...], 0.0)
        cls_ref[0, i, 0] = (
            jnp.dot(w2c_ref[...], hc, preferred_element_type=jnp.float32)
            + b2c_ref[...])

        hr = jnp.maximum(
            jnp.dot(w1r_ref[...], xb, preferred_element_type=jnp.float32)
            + b1r_ref[...], 0.0)
        off = (jnp.dot(w2r_ref[...], hr, preferred_element_type=jnp.float32)
               + b2r_ref[...])

        cx = (_PC_RANGE[0] + _STRIDE[0] * (xrow0 + i).astype(jnp.float32)) \
            + np.float32(_STRIDE[0] / 2.0)
        om = off[0:36]
        dec_main = jnp.where(is_xyz, om * mult + (coord_yz + cx * sel_x),
                             jnp.exp(om) * mult)
        c = jax.nn.sigmoid(off[36:42]) * cosang
        ang = jnp.arctan2(c, jnp.sqrt(1.0 - c * c))

        pieces = []
        for a in range(_A):
            pieces.append(dec_main[6 * a:6 * a + 6])
            pieces.append(ang[a:a + 1])
        box_ref[0, i, 0] = jnp.concatenate(pieces, axis=0)


def kernel(x, W1c, b1c, gc, bec, mc, vc, W2c, b2c, W1r, b1r, gr, ber, mr, vr, W2r, b2r):
    B = x.shape[0]
    xt = jnp.transpose(x, (0, 2, 4, 1, 3))

    sc = gc * jax.lax.rsqrt(vc + 1e-5)
    w1c_eff = W1c * sc[:, None]
    b1c_eff = (b1c * sc + (bec - mc * sc))[:, None]
    sr = gr * jax.lax.rsqrt(vr + 1e-5)
    w1r_eff = W1r * sr[:, None]
    b1r_eff = (b1r * sr + (ber - mr * sr))[:, None]
    perm = jnp.asarray(_PERM)

    grid = (B, _GX // _XB, _GZ)
    full = lambda shape: pl.BlockSpec(shape, lambda b, xi, zi: (0,) * len(shape))
    cls_t, box_t = pl.pallas_call(
        _head_kernel,
        grid=grid,
        in_specs=[
            pl.BlockSpec((1, _XB, 1, _IN, _GY), lambda b, xi, zi: (b, xi, zi, 0, 0)),
            full((_IN, _IN)), full((_IN, 1)), full((_CLS_C, _IN)), full((_CLS_C, 1)),
            full((_IN, _IN)), full((_IN, 1)), full((_REG_C, _IN)), full((_REG_C, 1)),
            full((36, 5)), full((_A, 1)),
        ],
        out_specs=[
            pl.BlockSpec((1, _XB, 1, _CLS_C, _GY), lambda b, xi, zi: (b, xi, zi, 0, 0)),
            pl.BlockSpec((1, _XB, 1, _REG_C, _GY), lambda b, xi, zi: (b, xi, zi, 0, 0)),
        ],
        out_shape=[
            jax.ShapeDtypeStruct((B, _GX, _GZ, _CLS_C, _GY), jnp.float32),
            jax.ShapeDtypeStruct((B, _GX, _GZ, _REG_C, _GY), jnp.float32),
        ],
        compiler_params=pltpu.CompilerParams(
            dimension_semantics=("parallel", "parallel", "parallel")),
    )(xt, w1c_eff, b1c_eff, W2c, b2c[:, None],
      w1r_eff, b1r_eff, W2r[perm], b2r[perm][:, None],
      jnp.asarray(_TAB), jnp.asarray(_COS))

    cls_out = jnp.transpose(cls_t, (0, 1, 4, 2, 3))
    box_out = jnp.transpose(box_t, (0, 1, 4, 2, 3))
    return cls_out, box_out


# speedup vs baseline: 6.8000x; 2.0736x over previous
import numpy as np
import jax
import jax.numpy as jnp
from jax.experimental import pallas as pl
from jax.experimental.pallas import tpu as pltpu

_IN = 64
_GX, _GY, _GZ = 180, 200, 5
_A = 6
_BBOX = 7
_CLS_C = _A * 4
_REG_C = _A * _BBOX
_XB = 18

_PC_RANGE = np.array([0.0, -40.0, -4.0, 72.0, 40.0, 4.0], dtype=np.float32)
_ANCHOR_SIZES = np.array([
    [3.9, 1.6, 1.56, 0.0],
    [3.9, 1.6, 1.56, 1.5707963],
    [0.8, 0.6, 1.73, 0.0],
    [0.8, 0.6, 1.73, 1.5707963],
    [1.76, 0.6, 1.73, 0.0],
    [1.76, 0.6, 1.73, 1.5707963],
], dtype=np.float32)
_STRIDE = (_PC_RANGE[3:] - _PC_RANGE[:3]) / np.array([_GX, _GY, _GZ], np.float32)

_PERM = np.array([7 * a + k for a in range(_A) for k in range(6)]
                 + [7 * a + 6 for a in range(_A)], dtype=np.int32)
_KM = np.tile(np.arange(6), _A)
_AM = np.repeat(np.arange(_A), 6)
_MULT = _ANCHOR_SIZES[_AM, _KM % 3].astype(np.float32)[:, None]
_IS_XYZ = (_KM < 3).astype(np.float32)[:, None]
_SEL_X = (_KM == 0).astype(np.float32)[:, None]
_SEL_Y = (_KM == 1).astype(np.float32)[:, None]
_SEL_Z = (_KM == 2).astype(np.float32)[:, None]
_COS = np.cos(_ANCHOR_SIZES[:, 3]).astype(np.float32)[:, None]
_TAB = np.concatenate([_MULT, _IS_XYZ, _SEL_X, _SEL_Y, _SEL_Z], axis=1)


def _head_kernel(x_ref, w1c_ref, b1c_ref, w2c_ref, b2c_ref,
                 w1r_ref, b1r_ref, w2r_ref, b2r_ref, tab_ref, cos_ref,
                 cls_ref, box_ref):
    zi = pl.program_id(1)
    xrow0 = pl.program_id(0) * _XB
    cz = (_PC_RANGE[2] + _STRIDE[2] * zi.astype(jnp.float32)) + np.float32(_STRIDE[2] / 2.0)
    iy = jax.lax.broadcasted_iota(jnp.int32, (1, _GY), 1).astype(jnp.float32)
    cy = (_PC_RANGE[1] + _STRIDE[1] * iy) + np.float32(_STRIDE[1] / 2.0)

    mult = tab_ref[:, 0:1]
    is_xyz = tab_ref[:, 1:2] > 0.5
    coord_yz = cy * tab_ref[:, 3:4] + cz * tab_ref[:, 4:5]
    sel_x = tab_ref[:, 2:3]
    cosang = cos_ref[...]

    nb = x_ref.shape[0]
    for i in range(_XB):
        cx = (_PC_RANGE[0] + _STRIDE[0] * (xrow0 + i).astype(jnp.float32)) \
            + np.float32(_STRIDE[0] / 2.0)
        dec_mains, angs = [], []
        for b in range(nb):
            xb = x_ref[b, i, 0]
            hc = jnp.maximum(
                jnp.dot(w1c_ref[...], xb, preferred_element_type=jnp.float32)
                + b1c_ref[...], 0.0)
            cls_ref[b, i, 0] = (
                jnp.dot(w2c_ref[...], hc, preferred_element_type=jnp.float32)
                + b2c_ref[...])

            hr = jnp.maximum(
                jnp.dot(w1r_ref[...], xb, preferred_element_type=jnp.float32)
                + b1r_ref[...], 0.0)
            off = (jnp.dot(w2r_ref[...], hr, preferred_element_type=jnp.float32)
                   + b2r_ref[...])

            om = off[0:36]
            dec_mains.append(jnp.where(
                is_xyz, om * mult + (coord_yz + cx * sel_x),
                jnp.exp(om) * mult)[:, None, :])
            c = jax.nn.sigmoid(off[36:42]) * cosang
            angs.append(jnp.arctan2(c, jnp.sqrt(1.0 - c * c))[:, None, :])

        sm = jnp.concatenate(dec_mains, axis=1)
        sa = jnp.concatenate(angs, axis=1)
        pieces = []
        for a in range(_A):
            pieces.append(sm[6 * a:6 * a + 6])
            pieces.append(sa[a:a + 1])
        box_ref[i, 0] = jnp.concatenate(pieces, axis=0)


def kernel(x, W1c, b1c, gc, bec, mc, vc, W2c, b2c, W1r, b1r, gr, ber, mr, vr, W2r, b2r):
    B = x.shape[0]
    xt = jnp.transpose(x, (0, 2, 4, 1, 3))

    sc = gc * jax.lax.rsqrt(vc + 1e-5)
    w1c_eff = W1c * sc[:, None]
    b1c_eff = (b1c * sc + (bec - mc * sc))[:, None]
    sr = gr * jax.lax.rsqrt(vr + 1e-5)
    w1r_eff = W1r * sr[:, None]
    b1r_eff = (b1r * sr + (ber - mr * sr))[:, None]
    perm = jnp.asarray(_PERM)

    grid = (_GX // _XB, _GZ)
    full = lambda shape: pl.BlockSpec(shape, lambda xi, zi: (0,) * len(shape))
    cls_t, box_t = pl.pallas_call(
        _head_kernel,
        grid=grid,
        in_specs=[
            pl.BlockSpec((B, _XB, 1, _IN, _GY), lambda xi, zi: (0, xi, zi, 0, 0)),
            full((_IN, _IN)), full((_IN, 1)), full((_CLS_C, _IN)), full((_CLS_C, 1)),
            full((_IN, _IN)), full((_IN, 1)), full((_REG_C, _IN)), full((_REG_C, 1)),
            full((36, 5)), full((_A, 1)),
        ],
        out_specs=[
            pl.BlockSpec((B, _XB, 1, _CLS_C, _GY), lambda xi, zi: (0, xi, zi, 0, 0)),
            pl.BlockSpec((_XB, 1, _REG_C, B, _GY), lambda xi, zi: (xi, zi, 0, 0, 0)),
        ],
        out_shape=[
            jax.ShapeDtypeStruct((B, _GX, _GZ, _CLS_C, _GY), jnp.float32),
            jax.ShapeDtypeStruct((_GX, _GZ, _REG_C, B, _GY), jnp.float32),
        ],
        compiler_params=pltpu.CompilerParams(
            dimension_semantics=("parallel", "parallel")),
    )(xt, w1c_eff, b1c_eff, W2c, b2c[:, None],
      w1r_eff, b1r_eff, W2r[perm], b2r[perm][:, None],
      jnp.asarray(_TAB), jnp.asarray(_COS))

    cls_out = jnp.transpose(cls_t, (0, 1, 4, 2, 3))
    box_out = jnp.transpose(box_t, (3, 0, 4, 1, 2))
    return cls_out, box_out
